# Initial kernel scaffold; baseline (speedup 1.0000x reference)
import jax
import jax.numpy as jnp
from jax.experimental import pallas as pl


def kernel(x, edge_index, edge_attr, batch, W1, b1, W2, b2, LW1, Lb1, LW2, Lb2):
    raise NotImplementedError("write your pallas kernel here")


# speedup vs baseline: 1.1528x; 1.1528x over previous
import functools

import jax
import jax.numpy as jnp
from jax import lax
from jax.experimental import pallas as pl
from jax.experimental.pallas import tpu as pltpu
from jax.experimental.pallas import tpu_sc as plsc

L = 16
NC = 2
NS = 16
NW = NC * NS
NB = 512
G = 16
DUMP = 128


def _cdiv(a, b):
    return (a + b - 1) // b


def _remap_dst(didx, ci, half, g):
    local = didx - ci * half
    oob = (local < 0) | (local >= half)
    dump = jnp.full((L,), half + g * L, jnp.int32) + lax.iota(jnp.int32, L)
    return jnp.where(oob, dump, local)


def _make_deg_kernel(Np, KW2):
    half = Np // 2
    HT = half // NS
    HT2 = (half + DUMP) // NS
    mesh = plsc.VectorSubcoreMesh(core_axis_name="c", subcore_axis_name="s")

    @functools.partial(
        pl.kernel, mesh=mesh,
        compiler_params=pltpu.CompilerParams(use_tc_tiling_on_sc=False,
                                             needs_layout_passes=False),
        out_type=jax.ShapeDtypeStruct((Np,), jnp.float32),
        scratch_types=[
            pltpu.VMEM_SHARED((half + DUMP,), jnp.float32),
            pltpu.VMEM((128,), jnp.int32),
            pltpu.VMEM((128,), jnp.float32),
            pltpu.VMEM((HT2,), jnp.float32),
        ],
    )
    def k(dstp, wp, zeros1, out, acc_sh, didx, wv, stage):
        ci = lax.axis_index("c")
        si = lax.axis_index("s")
        pltpu.sync_copy(zeros1.at[pl.ds(si * HT2, HT2)], stage)
        pltpu.sync_copy(stage, acc_sh.at[pl.ds(si * HT2, HT2)])
        plsc.subcore_barrier()

        def win_body(j, carry):
            pltpu.sync_copy(dstp.at[si, j], didx)
            pltpu.sync_copy(wp.at[si, j], wv)
            for g in range(8):
                gs = pl.ds(g * L, L)
                didx[gs] = _remap_dst(didx[gs], ci, half, g)
            pltpu.sync_copy(wv, acc_sh.at[didx], add=True)
            return carry

        lax.fori_loop(0, KW2, win_body, 0)
        plsc.subcore_barrier()
        pltpu.sync_copy(acc_sh.at[pl.ds(si * HT, HT)], stage.at[pl.ds(0, HT)])
        pltpu.sync_copy(stage.at[pl.ds(0, HT)],
                        out.at[pl.ds(ci * half + si * HT, HT)])

    return k


def _make_norm_kernel(Np, KW2):
    mesh = plsc.VectorSubcoreMesh(core_axis_name="c", subcore_axis_name="s")

    @functools.partial(
        pl.kernel, mesh=mesh,
        compiler_params=pltpu.CompilerParams(use_tc_tiling_on_sc=False,
                                             needs_layout_passes=False),
        out_type=jax.ShapeDtypeStruct((NS, KW2, 128), jnp.float32),
        scratch_types=[
            pltpu.VMEM((128,), jnp.int32),
            pltpu.VMEM((128,), jnp.int32),
            pltpu.VMEM((128,), jnp.float32),
            pltpu.VMEM((128,), jnp.float32),
            pltpu.VMEM((128,), jnp.float32),
            pltpu.SemaphoreType.DMA,
        ],
    )
    def k(disv, srcp, dstp, wp, normp, sidx, didx, wbuf, dsv, ddv, sem):
        ci = lax.axis_index("c")
        si = lax.axis_index("s")
        nwin = (KW2 + 1 - ci) // 2

        def win_body(j2, carry):
            j = 2 * j2 + ci
            pltpu.sync_copy(srcp.at[si, j], sidx)
            pltpu.sync_copy(dstp.at[si, j], didx)
            pltpu.sync_copy(wp.at[si, j], wbuf)
            pltpu.async_copy(disv.at[sidx], dsv, sem).wait()
            pltpu.async_copy(disv.at[didx], ddv, sem).wait()
            for g in range(8):
                gs = pl.ds(g * L, L)
                wbuf[gs] = (dsv[gs] * wbuf[gs]) * ddv[gs]
            pltpu.sync_copy(wbuf, normp.at[si, j])
            return carry

        lax.fori_loop(0, nwin, win_body, 0)

    return k


def _make_prop_kernel(Np, KW2, CH):
    half = Np // 2
    HT = half // NS
    HT2 = (half + DUMP) // NS
    mesh = plsc.VectorSubcoreMesh(core_axis_name="c", subcore_axis_name="s")

    @functools.partial(
        pl.kernel, mesh=mesh,
        compiler_params=pltpu.CompilerParams(use_tc_tiling_on_sc=False,
                                             needs_layout_passes=False),
        out_type=jax.ShapeDtypeStruct((CH, Np, L), jnp.float32),
        scratch_types=[
            pltpu.VMEM_SHARED((half + DUMP, L), jnp.float32),
            pltpu.VMEM((128,), jnp.int32),
            pltpu.VMEM((128,), jnp.int32),
            pltpu.VMEM((128,), jnp.float32),
            pltpu.VMEM((128, L), jnp.float32),
            pltpu.VMEM((HT2, L), jnp.float32),
            pltpu.VMEM((HT2, L), jnp.float32),
            pltpu.SemaphoreType.DMA,
        ],
    )
    def k(tbl, srcp, dstp, normp, zeros2, out,
          acc_sh, sidx, didx, wbuf, rows, stage, zbuf, sem):
        ci = lax.axis_index("c")
        si = lax.axis_index("s")
        zsl = pl.ds(si * HT2, HT2)
        pltpu.sync_copy(zeros2.at[zsl], zbuf)

        def chunk_body(c, carry):
            pltpu.sync_copy(zbuf, acc_sh.at[zsl])
            plsc.subcore_barrier()
            base = c * Np

            def win_body(j, wcarry):
                pltpu.sync_copy(srcp.at[si, j], sidx)
                pltpu.sync_copy(dstp.at[si, j], didx)
                pltpu.sync_copy(normp.at[si, j], wbuf)
                for g in range(8):
                    gs = pl.ds(g * L, L)
                    sidx[gs] = sidx[gs] + base
                    didx[gs] = _remap_dst(didx[gs], ci, half, g)
                pltpu.async_copy(tbl.at[sidx], rows, sem).wait()
                for e in range(128):
                    wsp = plsc.load_gather(
                        wbuf, [jnp.full((L,), e, jnp.int32)])
                    rows[e, :] = rows[e, :] * wsp
                pltpu.sync_copy(rows, acc_sh.at[didx], add=True)
                return wcarry

            lax.fori_loop(0, KW2, win_body, 0)
            plsc.subcore_barrier()
            pltpu.sync_copy(acc_sh.at[pl.ds(si * HT, HT)],
                            stage.at[pl.ds(0, HT)])
            pltpu.sync_copy(stage.at[pl.ds(0, HT)],
                            out.at[c, pl.ds(ci * half + si * HT, HT)])
            plsc.subcore_barrier()
            return carry

        lax.fori_loop(0, CH, chunk_body, 0)

    return k


def _silu(x):
    return x / (1.0 + jnp.exp(-x))


def _make_pre_kernel(Np, D1P, H1P, interpret=False):
    GR = Np // NB
    CH = H1P // L

    def body(d0, x16, W1p, dis_ref, sw_ref, xw_ref):
        deg = d0[0, 0, :] + 1.0
        dis = jnp.where(deg > 0,
                        lax.rsqrt(jnp.maximum(deg, 1e-12)),
                        0.0)
        dis_ref[0, 0, :] = dis
        sw_ref[0, 0, :] = dis * dis
        xw = jnp.dot(x16[...], W1p[...], preferred_element_type=jnp.float32)
        for c in range(CH):
            xw_ref[c] = xw[:, c * L:(c + 1) * L]

    return pl.pallas_call(
        body,
        grid=(GR,),
        in_specs=[
            pl.BlockSpec((1, 1, NB), lambda i: (i, 0, 0)),
            pl.BlockSpec((NB, D1P), lambda i: (i, 0)),
            pl.BlockSpec((D1P, H1P), lambda i: (0, 0)),
        ],
        out_specs=[
            pl.BlockSpec((1, 1, NB), lambda i: (i, 0, 0)),
            pl.BlockSpec((1, 1, NB), lambda i: (i, 0, 0)),
            pl.BlockSpec((CH, NB, L), lambda i: (0, i, 0)),
        ],
        out_shape=[
            jax.ShapeDtypeStruct((GR, 1, NB), jnp.float32),
            jax.ShapeDtypeStruct((GR, 1, NB), jnp.float32),
            jax.ShapeDtypeStruct((CH, Np, L), jnp.float32),
        ],
        interpret=interpret,
    )


def _make_layer1_kernel(Np, H1P, H2P, interpret=False):
    GR = Np // NB
    CH = H1P // L
    C2 = H2P // L

    def body(a0, xw1, sw2, b1p, W2p, xw2_ref):
        sw = sw2[0, 0, :]
        conv = jnp.concatenate(
            [a0[c] + xw1[c] * sw[:, None] for c in range(CH)], axis=1)
        h1 = _silu(conv + b1p[...])
        xw2 = jnp.dot(h1, W2p[...], preferred_element_type=jnp.float32)
        for c in range(C2):
            xw2_ref[c] = xw2[:, c * L:(c + 1) * L]

    return pl.pallas_call(
        body,
        grid=(GR,),
        in_specs=[
            pl.BlockSpec((CH, NB, L), lambda i: (0, i, 0)),
            pl.BlockSpec((CH, NB, L), lambda i: (0, i, 0)),
            pl.BlockSpec((1, 1, NB), lambda i: (i, 0, 0)),
            pl.BlockSpec((1, H1P), lambda i: (0, 0)),
            pl.BlockSpec((H1P, H2P), lambda i: (0, 0)),
        ],
        out_specs=pl.BlockSpec((C2, NB, L), lambda i: (0, i, 0)),
        out_shape=jax.ShapeDtypeStruct((C2, Np, L), jnp.float32),
        interpret=interpret,
    )


def _make_final_kernel(Np, H2P, HM, interpret=False):
    GR = Np // NB
    C2 = H2P // L

    def body(a0, xw2, sw2, bat2, b2p, LW1p, Lb1, LW2p, Lb2,
             out_ref, pooled_ref, cnt_ref):
        i = pl.program_id(0)

        @pl.when(i == 0)
        def _():
            pooled_ref[...] = jnp.zeros_like(pooled_ref)
            cnt_ref[...] = jnp.zeros_like(cnt_ref)

        sw = sw2[0, 0, :]
        conv = jnp.concatenate(
            [a0[c] + xw2[c] * sw[:, None] for c in range(C2)], axis=1)
        h2 = _silu(conv + b2p[...])
        b = bat2[0, 0, :]
        seg = lax.broadcasted_iota(jnp.int32, (NB, G), 1)
        onehot = (b[:, None] == seg).astype(jnp.float32)
        pooled_ref[...] += lax.dot_general(
            onehot, h2, (((0,), (0,)), ((), ())),
            preferred_element_type=jnp.float32,
            precision=lax.Precision.HIGHEST)
        cnt_ref[...] += jnp.sum(onehot, axis=0)[None, :]

        @pl.when(i == GR - 1)
        def _():
            cnt = jnp.maximum(cnt_ref[0, :], 1.0)
            pooled = pooled_ref[...] / cnt[:, None]
            hh = _silu(jnp.dot(pooled, LW1p[...],
                               preferred_element_type=jnp.float32) + Lb1[...])
            o = jnp.dot(hh, LW2p[...],
                        preferred_element_type=jnp.float32) + Lb2[0, 0]
            out_ref[...] = o[:, 0][None, :]

    return pl.pallas_call(
        body,
        grid=(GR,),
        in_specs=[
            pl.BlockSpec((C2, NB, L), lambda i: (0, i, 0)),
            pl.BlockSpec((C2, NB, L), lambda i: (0, i, 0)),
            pl.BlockSpec((1, 1, NB), lambda i: (i, 0, 0)),
            pl.BlockSpec((1, 1, NB), lambda i: (i, 0, 0)),
            pl.BlockSpec((1, H2P), lambda i: (0, 0)),
            pl.BlockSpec((H2P, HM), lambda i: (0, 0)),
            pl.BlockSpec((1, HM), lambda i: (0, 0)),
            pl.BlockSpec((HM, 128), lambda i: (0, 0)),
            pl.BlockSpec((1, 1), lambda i: (0, 0)),
        ],
        out_specs=pl.BlockSpec((1, G), lambda i: (0, 0)),
        out_shape=jax.ShapeDtypeStruct((1, G), jnp.float32),
        scratch_shapes=[
            pltpu.VMEM((G, H2P), jnp.float32),
            pltpu.VMEM((1, G), jnp.float32),
        ],
        interpret=interpret,
    )


def kernel(x, edge_index, edge_attr, batch,
           W1, b1, W2, b2, LW1, Lb1, LW2, Lb2):
    N, D = x.shape
    E = edge_index.shape[1]
    H1 = W1.shape[1]
    H2 = W2.shape[1]
    HM = LW1.shape[1]

    Np = _cdiv(N, 2 * NB) * 2 * NB
    KW2 = _cdiv(E, NS * 128)
    Ep = NS * KW2 * 128
    H1P = _cdiv(H1, L) * L
    H2P = _cdiv(H2, L) * L
    D1P = L

    f32 = jnp.float32
    i32 = jnp.int32

    src = edge_index[0].astype(i32)
    dst = edge_index[1].astype(i32)
    w = edge_attr.astype(f32)
    pad = Ep - E
    spread = (jnp.arange(pad, dtype=i32) % N)
    srcp = jnp.concatenate([src, spread]).reshape(NS, KW2, 128)
    dstp = jnp.concatenate([dst, spread]).reshape(NS, KW2, 128)
    wp = jnp.concatenate([w, jnp.zeros((pad,), f32)]).reshape(NS, KW2, 128)

    x16 = jnp.pad(x.astype(f32), ((0, Np - N), (0, D1P - D)))
    batp = jnp.pad(batch.astype(i32), (0, Np - N), constant_values=-1)
    zeros1 = jnp.zeros((Np,), f32)
    zeros2 = jnp.zeros((Np, L), f32)

    W1p = jnp.pad(W1.astype(f32), ((0, D1P - D), (0, H1P - H1)))
    b1p = jnp.pad(b1.astype(f32), (0, H1P - H1)).reshape(1, H1P)
    W2p = jnp.pad(W2.astype(f32), ((0, H1P - H1), (0, H2P - H2)))
    b2p = jnp.pad(b2.astype(f32), (0, H2P - H2)).reshape(1, H2P)
    LW1p = jnp.pad(LW1.astype(f32), ((0, H2P - H2), (0, 0)))
    Lb1r = Lb1.astype(f32).reshape(1, HM)
    LW2p = jnp.pad(LW2.astype(f32), ((0, 0), (0, 128 - LW2.shape[1])))
    Lb2r = Lb2.astype(f32).reshape(1, 1)

    GR = Np // NB

    deg = _make_deg_kernel(Np, KW2)(dstp, wp, zeros1)

    dis2, sw2, xw1 = _make_pre_kernel(Np, D1P, H1P)(
        deg.reshape(GR, 1, NB), x16, W1p)

    normp = _make_norm_kernel(Np, KW2)(dis2.reshape(Np), srcp, dstp, wp)

    a1 = _make_prop_kernel(Np, KW2, H1P // L)(
        xw1.reshape((H1P // L) * Np, L), srcp, dstp, normp, zeros2)

    xw2 = _make_layer1_kernel(Np, H1P, H2P)(a1, xw1, sw2, b1p, W2p)

    a2 = _make_prop_kernel(Np, KW2, H2P // L)(
        xw2.reshape((H2P // L) * Np, L), srcp, dstp, normp, zeros2)

    out2 = _make_final_kernel(Np, H2P, HM)(
        a2, xw2, sw2, batp.reshape(GR, 1, NB),
        b2p, LW1p, Lb1r, LW2p, Lb2r)

    return out2.reshape(G)


# speedup vs baseline: 1.9974x; 1.7326x over previous
import functools

import jax
import jax.numpy as jnp
from jax import lax
from jax.experimental import pallas as pl
from jax.experimental.pallas import tpu as pltpu
from jax.experimental.pallas import tpu_sc as plsc

L = 16
NC = 2
NS = 16
NW = NC * NS
NB = 512
G = 16
DUMP = 128


def _cdiv(a, b):
    return (a + b - 1) // b


def _remap_dst(didx, ci, half, g):
    local = didx - ci * half
    oob = (local < 0) | (local >= half)
    dump = jnp.full((L,), half + g * L, jnp.int32) + lax.iota(jnp.int32, L)
    return jnp.where(oob, dump, local)


def _make_deg_kernel(Np, KW2):
    half = Np // 2
    HT = half // NS
    HT2 = (half + DUMP) // NS
    mesh = plsc.VectorSubcoreMesh(core_axis_name="c", subcore_axis_name="s")

    @functools.partial(
        pl.kernel, mesh=mesh,
        compiler_params=pltpu.CompilerParams(use_tc_tiling_on_sc=False,
                                             needs_layout_passes=False),
        out_type=jax.ShapeDtypeStruct((Np,), jnp.float32),
        scratch_types=[
            pltpu.VMEM_SHARED((half + DUMP,), jnp.float32),
            pltpu.VMEM((128,), jnp.int32),
            pltpu.VMEM((128,), jnp.float32),
            pltpu.VMEM((HT2,), jnp.float32),
        ],
    )
    def k(dstp, wp, zeros1, out, acc_sh, didx, wv, stage):
        ci = lax.axis_index("c")
        si = lax.axis_index("s")
        pltpu.sync_copy(zeros1.at[pl.ds(si * HT2, HT2)], stage)
        pltpu.sync_copy(stage, acc_sh.at[pl.ds(si * HT2, HT2)])
        plsc.subcore_barrier()

        def win_body(j, carry):
            pltpu.sync_copy(dstp.at[si, j], didx)
            pltpu.sync_copy(wp.at[si, j], wv)
            for g in range(8):
                gs = pl.ds(g * L, L)
                didx[gs] = _remap_dst(didx[gs], ci, half, g)
            pltpu.sync_copy(wv, acc_sh.at[didx], add=True)
            return carry

        lax.fori_loop(0, KW2, win_body, 0)
        plsc.subcore_barrier()
        pltpu.sync_copy(acc_sh.at[pl.ds(si * HT, HT)], stage.at[pl.ds(0, HT)])
        pltpu.sync_copy(stage.at[pl.ds(0, HT)],
                        out.at[pl.ds(ci * half + si * HT, HT)])

    return k


def _make_norm_kernel(Np, KW2):
    mesh = plsc.VectorSubcoreMesh(core_axis_name="c", subcore_axis_name="s")

    @functools.partial(
        pl.kernel, mesh=mesh,
        compiler_params=pltpu.CompilerParams(use_tc_tiling_on_sc=False,
                                             needs_layout_passes=False),
        out_type=jax.ShapeDtypeStruct((NS, KW2, 128), jnp.float32),
        scratch_types=[
            pltpu.VMEM((128,), jnp.int32),
            pltpu.VMEM((128,), jnp.int32),
            pltpu.VMEM((128,), jnp.float32),
            pltpu.VMEM((128,), jnp.float32),
            pltpu.VMEM((128,), jnp.float32),
            pltpu.SemaphoreType.DMA,
        ],
    )
    def k(disv, srcp, dstp, wp, normp, sidx, didx, wbuf, dsv, ddv, sem):
        ci = lax.axis_index("c")
        si = lax.axis_index("s")
        nwin = (KW2 + 1 - ci) // 2

        def win_body(j2, carry):
            j = 2 * j2 + ci
            pltpu.sync_copy(srcp.at[si, j], sidx)
            pltpu.sync_copy(dstp.at[si, j], didx)
            pltpu.sync_copy(wp.at[si, j], wbuf)
            pltpu.async_copy(disv.at[sidx], dsv, sem).wait()
            pltpu.async_copy(disv.at[didx], ddv, sem).wait()
            for g in range(8):
                gs = pl.ds(g * L, L)
                wbuf[gs] = (dsv[gs] * wbuf[gs]) * ddv[gs]
            pltpu.sync_copy(wbuf, normp.at[si, j])
            return carry

        lax.fori_loop(0, nwin, win_body, 0)

    return k


def _make_prop_kernel(Np, KW3, CH):
    half = Np // 2
    HT = half // NS
    HT2 = (half + DUMP) // NS
    mesh = plsc.VectorSubcoreMesh(core_axis_name="c", subcore_axis_name="s")

    @functools.partial(
        pl.kernel, mesh=mesh,
        compiler_params=pltpu.CompilerParams(use_tc_tiling_on_sc=False,
                                             needs_layout_passes=False),
        out_type=jax.ShapeDtypeStruct((CH, Np, L), jnp.float32),
        scratch_types=[
            pltpu.VMEM_SHARED((half + DUMP, L), jnp.float32),
            pltpu.VMEM((4, 128), jnp.int32),
            pltpu.VMEM((4, 128), jnp.int32),
            pltpu.VMEM((512,), jnp.float32),
            pltpu.VMEM((512, L), jnp.float32),
            pltpu.VMEM((HT2, L), jnp.float32),
            pltpu.VMEM((HT2, L), jnp.float32),
            pltpu.SemaphoreType.DMA,
            pltpu.SemaphoreType.DMA,
        ],
    )
    def k(tbl, srcp, dstp, normp, zeros2, out,
          acc_sh, sidx, didx, wbuf, rows, stage, zbuf, gsem, ssem):
        ci = lax.axis_index("c")
        si = lax.axis_index("s")
        zsl = pl.ds(si * HT2, HT2)
        pltpu.sync_copy(zeros2.at[zsl], zbuf)

        def chunk_body(c, carry):
            pltpu.sync_copy(zbuf, acc_sh.at[zsl])
            plsc.subcore_barrier()
            base = c * Np

            def win_body(j, wcarry):
                pltpu.sync_copy(srcp.at[si, j], sidx)
                pltpu.sync_copy(dstp.at[si, j], didx)
                pltpu.sync_copy(normp.at[si, j], wbuf)
                for k4 in range(4):
                    for g in range(8):
                        gs = pl.ds(g * L, L)
                        sidx[k4, gs] = sidx[k4, gs] + base
                        didx[k4, gs] = _remap_dst(didx[k4, gs], ci, half, g)
                gets = [
                    pltpu.async_copy(tbl.at[sidx.at[k4]],
                                     rows.at[pl.ds(k4 * 128, 128)], gsem)
                    for k4 in range(4)
                ]
                for d in gets:
                    d.wait()
                puts = []
                for k4 in range(4):
                    for e in range(k4 * 128, k4 * 128 + 128):
                        wsp = plsc.load_gather(
                            wbuf, [jnp.full((L,), e, jnp.int32)])
                        rows[e, :] = rows[e, :] * wsp
                    puts.append(pltpu.async_copy(
                        rows.at[pl.ds(k4 * 128, 128)],
                        acc_sh.at[didx.at[k4]], ssem, add=True))
                for d in puts:
                    d.wait()
                return wcarry

            lax.fori_loop(0, KW3, win_body, 0)
            plsc.subcore_barrier()
            pltpu.sync_copy(acc_sh.at[pl.ds(si * HT, HT)],
                            stage.at[pl.ds(0, HT)])
            pltpu.sync_copy(stage.at[pl.ds(0, HT)],
                            out.at[c, pl.ds(ci * half + si * HT, HT)])
            plsc.subcore_barrier()
            return carry

        lax.fori_loop(0, CH, chunk_body, 0)

    return k


def _silu(x):
    return x / (1.0 + jnp.exp(-x))


def _make_pre_kernel(Np, D1P, H1P, interpret=False):
    GR = Np // NB
    CH = H1P // L

    def body(d0, x16, W1p, dis_ref, sw_ref, xw_ref):
        deg = d0[0, 0, :] + 1.0
        dis = jnp.where(deg > 0,
                        lax.rsqrt(jnp.maximum(deg, 1e-12)),
                        0.0)
        dis_ref[0, 0, :] = dis
        sw_ref[0, 0, :] = dis * dis
        xw = jnp.dot(x16[...], W1p[...], preferred_element_type=jnp.float32)
        for c in range(CH):
            xw_ref[c] = xw[:, c * L:(c + 1) * L]

    return pl.pallas_call(
        body,
        grid=(GR,),
        in_specs=[
            pl.BlockSpec((1, 1, NB), lambda i: (i, 0, 0)),
            pl.BlockSpec((NB, D1P), lambda i: (i, 0)),
            pl.BlockSpec((D1P, H1P), lambda i: (0, 0)),
        ],
        out_specs=[
            pl.BlockSpec((1, 1, NB), lambda i: (i, 0, 0)),
            pl.BlockSpec((1, 1, NB), lambda i: (i, 0, 0)),
            pl.BlockSpec((CH, NB, L), lambda i: (0, i, 0)),
        ],
        out_shape=[
            jax.ShapeDtypeStruct((GR, 1, NB), jnp.float32),
            jax.ShapeDtypeStruct((GR, 1, NB), jnp.float32),
            jax.ShapeDtypeStruct((CH, Np, L), jnp.float32),
        ],
        interpret=interpret,
    )


def _make_layer1_kernel(Np, H1P, H2P, interpret=False):
    GR = Np // NB
    CH = H1P // L
    C2 = H2P // L

    def body(a0, xw1, sw2, b1p, W2p, xw2_ref):
        sw = sw2[0, 0, :]
        conv = jnp.concatenate(
            [a0[c] + xw1[c] * sw[:, None] for c in range(CH)], axis=1)
        h1 = _silu(conv + b1p[...])
        xw2 = jnp.dot(h1, W2p[...], preferred_element_type=jnp.float32)
        for c in range(C2):
            xw2_ref[c] = xw2[:, c * L:(c + 1) * L]

    return pl.pallas_call(
        body,
        grid=(GR,),
        in_specs=[
            pl.BlockSpec((CH, NB, L), lambda i: (0, i, 0)),
            pl.BlockSpec((CH, NB, L), lambda i: (0, i, 0)),
            pl.BlockSpec((1, 1, NB), lambda i: (i, 0, 0)),
            pl.BlockSpec((1, H1P), lambda i: (0, 0)),
            pl.BlockSpec((H1P, H2P), lambda i: (0, 0)),
        ],
        out_specs=pl.BlockSpec((C2, NB, L), lambda i: (0, i, 0)),
        out_shape=jax.ShapeDtypeStruct((C2, Np, L), jnp.float32),
        interpret=interpret,
    )


def _make_final_kernel(Np, H2P, HM, interpret=False):
    GR = Np // NB
    C2 = H2P // L

    def body(a0, xw2, sw2, bat2, b2p, LW1p, Lb1, LW2p, Lb2,
             out_ref, pooled_ref, cnt_ref):
        i = pl.program_id(0)

        @pl.when(i == 0)
        def _():
            pooled_ref[...] = jnp.zeros_like(pooled_ref)
            cnt_ref[...] = jnp.zeros_like(cnt_ref)

        sw = sw2[0, 0, :]
        conv = jnp.concatenate(
            [a0[c] + xw2[c] * sw[:, None] for c in range(C2)], axis=1)
        h2 = _silu(conv + b2p[...])
        b = bat2[0, 0, :]
        seg = lax.broadcasted_iota(jnp.int32, (NB, G), 1)
        onehot = (b[:, None] == seg).astype(jnp.float32)
        pooled_ref[...] += lax.dot_general(
            onehot, h2, (((0,), (0,)), ((), ())),
            preferred_element_type=jnp.float32,
            precision=lax.Precision.HIGHEST)
        cnt_ref[...] += jnp.sum(onehot, axis=0)[None, :]

        @pl.when(i == GR - 1)
        def _():
            cnt = jnp.maximum(cnt_ref[0, :], 1.0)
            pooled = pooled_ref[...] / cnt[:, None]
            hh = _silu(jnp.dot(pooled, LW1p[...],
                               preferred_element_type=jnp.float32) + Lb1[...])
            o = jnp.dot(hh, LW2p[...],
                        preferred_element_type=jnp.float32) + Lb2[0, 0]
            out_ref[...] = o[:, 0][None, :]

    return pl.pallas_call(
        body,
        grid=(GR,),
        in_specs=[
            pl.BlockSpec((C2, NB, L), lambda i: (0, i, 0)),
            pl.BlockSpec((C2, NB, L), lambda i: (0, i, 0)),
            pl.BlockSpec((1, 1, NB), lambda i: (i, 0, 0)),
            pl.BlockSpec((1, 1, NB), lambda i: (i, 0, 0)),
            pl.BlockSpec((1, H2P), lambda i: (0, 0)),
            pl.BlockSpec((H2P, HM), lambda i: (0, 0)),
            pl.BlockSpec((1, HM), lambda i: (0, 0)),
            pl.BlockSpec((HM, 128), lambda i: (0, 0)),
            pl.BlockSpec((1, 1), lambda i: (0, 0)),
        ],
        out_specs=pl.BlockSpec((1, G), lambda i: (0, 0)),
        out_shape=jax.ShapeDtypeStruct((1, G), jnp.float32),
        scratch_shapes=[
            pltpu.VMEM((G, H2P), jnp.float32),
            pltpu.VMEM((1, G), jnp.float32),
        ],
        interpret=interpret,
    )


def kernel(x, edge_index, edge_attr, batch,
           W1, b1, W2, b2, LW1, Lb1, LW2, Lb2):
    N, D = x.shape
    E = edge_index.shape[1]
    H1 = W1.shape[1]
    H2 = W2.shape[1]
    HM = LW1.shape[1]

    Np = _cdiv(N, 2 * NB) * 2 * NB
    KW3 = _cdiv(E, NS * 512)
    KW2 = 4 * KW3
    Ep = NS * KW2 * 128
    H1P = _cdiv(H1, L) * L
    H2P = _cdiv(H2, L) * L
    D1P = L

    f32 = jnp.float32
    i32 = jnp.int32

    src = edge_index[0].astype(i32)
    dst = edge_index[1].astype(i32)
    w = edge_attr.astype(f32)
    pad = Ep - E
    spread = (jnp.arange(pad, dtype=i32) % N)
    srcp = jnp.concatenate([src, spread]).reshape(NS, KW2, 128)
    dstp = jnp.concatenate([dst, spread]).reshape(NS, KW2, 128)
    wp = jnp.concatenate([w, jnp.zeros((pad,), f32)]).reshape(NS, KW2, 128)

    x16 = jnp.pad(x.astype(f32), ((0, Np - N), (0, D1P - D)))
    batp = jnp.pad(batch.astype(i32), (0, Np - N), constant_values=-1)
    zeros1 = jnp.zeros((Np,), f32)
    zeros2 = jnp.zeros((Np, L), f32)

    W1p = jnp.pad(W1.astype(f32), ((0, D1P - D), (0, H1P - H1)))
    b1p = jnp.pad(b1.astype(f32), (0, H1P - H1)).reshape(1, H1P)
    W2p = jnp.pad(W2.astype(f32), ((0, H1P - H1), (0, H2P - H2)))
    b2p = jnp.pad(b2.astype(f32), (0, H2P - H2)).reshape(1, H2P)
    LW1p = jnp.pad(LW1.astype(f32), ((0, H2P - H2), (0, 0)))
    Lb1r = Lb1.astype(f32).reshape(1, HM)
    LW2p = jnp.pad(LW2.astype(f32), ((0, 0), (0, 128 - LW2.shape[1])))
    Lb2r = Lb2.astype(f32).reshape(1, 1)

    GR = Np // NB

    deg = _make_deg_kernel(Np, KW2)(dstp, wp, zeros1)

    dis2, sw2, xw1 = _make_pre_kernel(Np, D1P, H1P)(
        deg.reshape(GR, 1, NB), x16, W1p)

    normp = _make_norm_kernel(Np, KW2)(dis2.reshape(Np), srcp, dstp, wp)

    srcp4 = srcp.reshape(NS, KW3, 4, 128)
    dstp4 = dstp.reshape(NS, KW3, 4, 128)
    normp4 = normp.reshape(NS, KW3, 512)

    a1 = _make_prop_kernel(Np, KW3, H1P // L)(
        xw1.reshape((H1P // L) * Np, L), srcp4, dstp4, normp4, zeros2)

    xw2 = _make_layer1_kernel(Np, H1P, H2P)(a1, xw1, sw2, b1p, W2p)

    a2 = _make_prop_kernel(Np, KW3, H2P // L)(
        xw2.reshape((H2P // L) * Np, L), srcp4, dstp4, normp4, zeros2)

    out2 = _make_final_kernel(Np, H2P, HM)(
        a2, xw2, sw2, batp.reshape(GR, 1, NB),
        b2p, LW1p, Lb1r, LW2p, Lb2r)

    return out2.reshape(G)


# speedup vs baseline: 2.0512x; 1.0269x over previous
import functools

import jax
import jax.numpy as jnp
from jax import lax
from jax.experimental import pallas as pl
from jax.experimental.pallas import tpu as pltpu
from jax.experimental.pallas import tpu_sc as plsc

L = 16
NC = 2
NS = 16
NW = NC * NS
NB = 512
G = 16
DUMP = 128


def _cdiv(a, b):
    return (a + b - 1) // b


def _remap_dst(didx, ci, half, g):
    local = didx - ci * half
    oob = (local < 0) | (local >= half)
    dump = jnp.full((L,), half + g * L, jnp.int32) + lax.iota(jnp.int32, L)
    return jnp.where(oob, dump, local)


def _make_deg_kernel(Np, KW2):
    half = Np // 2
    HT = half // NS
    HT2 = (half + DUMP) // NS
    mesh = plsc.VectorSubcoreMesh(core_axis_name="c", subcore_axis_name="s")

    @functools.partial(
        pl.kernel, mesh=mesh,
        compiler_params=pltpu.CompilerParams(use_tc_tiling_on_sc=False,
                                             needs_layout_passes=False),
        out_type=jax.ShapeDtypeStruct((Np,), jnp.float32),
        scratch_types=[
            pltpu.VMEM_SHARED((half + DUMP,), jnp.float32),
            pltpu.VMEM((128,), jnp.int32),
            pltpu.VMEM((128,), jnp.float32),
            pltpu.VMEM((HT2,), jnp.float32),
        ],
    )
    def k(dstp, wp, zeros1, out, acc_sh, didx, wv, stage):
        ci = lax.axis_index("c")
        si = lax.axis_index("s")
        pltpu.sync_copy(zeros1.at[pl.ds(si * HT2, HT2)], stage)
        pltpu.sync_copy(stage, acc_sh.at[pl.ds(si * HT2, HT2)])
        plsc.subcore_barrier()

        def win_body(j, carry):
            pltpu.sync_copy(dstp.at[si, j], didx)
            pltpu.sync_copy(wp.at[si, j], wv)
            for g in range(8):
                gs = pl.ds(g * L, L)
                didx[gs] = _remap_dst(didx[gs], ci, half, g)
            pltpu.sync_copy(wv, acc_sh.at[didx], add=True)
            return carry

        lax.fori_loop(0, KW2, win_body, 0)
        plsc.subcore_barrier()
        pltpu.sync_copy(acc_sh.at[pl.ds(si * HT, HT)], stage.at[pl.ds(0, HT)])
        pltpu.sync_copy(stage.at[pl.ds(0, HT)],
                        out.at[pl.ds(ci * half + si * HT, HT)])

    return k


def _make_norm_kernel(Np, KW2):
    mesh = plsc.VectorSubcoreMesh(core_axis_name="c", subcore_axis_name="s")

    @functools.partial(
        pl.kernel, mesh=mesh,
        compiler_params=pltpu.CompilerParams(use_tc_tiling_on_sc=False,
                                             needs_layout_passes=False),
        out_type=jax.ShapeDtypeStruct((NS, KW2, 128), jnp.float32),
        scratch_types=[
            pltpu.VMEM((128,), jnp.int32),
            pltpu.VMEM((128,), jnp.int32),
            pltpu.VMEM((128,), jnp.float32),
            pltpu.VMEM((128,), jnp.float32),
            pltpu.VMEM((128,), jnp.float32),
            pltpu.SemaphoreType.DMA,
        ],
    )
    def k(disv, srcp, dstp, wp, normp, sidx, didx, wbuf, dsv, ddv, sem):
        ci = lax.axis_index("c")
        si = lax.axis_index("s")
        nwin = (KW2 + 1 - ci) // 2

        def win_body(j2, carry):
            j = 2 * j2 + ci
            pltpu.sync_copy(srcp.at[si, j], sidx)
            pltpu.sync_copy(dstp.at[si, j], didx)
            pltpu.sync_copy(wp.at[si, j], wbuf)
            pltpu.async_copy(disv.at[sidx], dsv, sem).wait()
            pltpu.async_copy(disv.at[didx], ddv, sem).wait()
            for g in range(8):
                gs = pl.ds(g * L, L)
                wbuf[gs] = (dsv[gs] * wbuf[gs]) * ddv[gs]
            pltpu.sync_copy(wbuf, normp.at[si, j])
            return carry

        lax.fori_loop(0, nwin, win_body, 0)

    return k


def _make_prop_kernel(Np, KW3, CH):
    half = Np // 2
    HT = half // NS
    HT2 = (half + DUMP) // NS
    mesh = plsc.VectorSubcoreMesh(core_axis_name="c", subcore_axis_name="s")

    @functools.partial(
        pl.kernel, mesh=mesh,
        compiler_params=pltpu.CompilerParams(use_tc_tiling_on_sc=False,
                                             needs_layout_passes=False),
        out_type=jax.ShapeDtypeStruct((CH, Np, L), jnp.float32),
        scratch_types=[
            pltpu.VMEM_SHARED((half + DUMP, L), jnp.float32),
            pltpu.VMEM((2, 4, 128), jnp.int32),
            pltpu.VMEM((2, 4, 128), jnp.int32),
            pltpu.VMEM((2, 512), jnp.float32),
            pltpu.VMEM((2, 512, L), jnp.float32),
            pltpu.VMEM((HT2, L), jnp.float32),
            pltpu.VMEM((HT2, L), jnp.float32),
            pltpu.SemaphoreType.DMA,
            pltpu.SemaphoreType.DMA,
            pltpu.SemaphoreType.DMA,
            pltpu.SemaphoreType.DMA,
        ],
    )
    def k(tbl, srcp, dstp, normp, zeros2, out,
          acc_sh, sidx, didx, wbuf, rows, stage, zbuf,
          gsem0, gsem1, ssem0, ssem1):
        ci = lax.axis_index("c")
        si = lax.axis_index("s")
        zsl = pl.ds(si * HT2, HT2)
        pltpu.sync_copy(zeros2.at[zsl], zbuf)
        gsems = (gsem0, gsem1)
        ssems = (ssem0, ssem1)

        def stage_fire(j, b, base):
            pltpu.sync_copy(srcp.at[si, j], sidx.at[b])
            pltpu.sync_copy(dstp.at[si, j], didx.at[b])
            pltpu.sync_copy(normp.at[si, j], wbuf.at[b])
            for k4 in range(4):
                for g in range(8):
                    gs = pl.ds(g * L, L)
                    sidx[b, k4, gs] = sidx[b, k4, gs] + base
                    didx[b, k4, gs] = _remap_dst(didx[b, k4, gs],
                                                 ci, half, g)
                pltpu.async_copy(tbl.at[sidx.at[b, k4]],
                                 rows.at[b, pl.ds(k4 * 128, 128)], gsems[b])

        def drain_scale_fire(b):
            for k4 in range(4):
                pltpu.make_async_copy(
                    tbl.at[sidx.at[b, k4]],
                    rows.at[b, pl.ds(k4 * 128, 128)], gsems[b]).wait()
            puts = []
            for k4 in range(4):
                for e in range(k4 * 128, k4 * 128 + 128):
                    wsp = plsc.load_gather(
                        wbuf.at[b], [jnp.full((L,), e, jnp.int32)])
                    rows[b, e, :] = rows[b, e, :] * wsp
                puts.append(pltpu.async_copy(
                    rows.at[b, pl.ds(k4 * 128, 128)],
                    acc_sh.at[didx.at[b, k4]], ssems[b], add=True))
            return puts

        def chunk_body(c, carry):
            pltpu.sync_copy(zbuf, acc_sh.at[zsl])
            plsc.subcore_barrier()
            base = c * Np
            stage_fire(0, 0, base)
            stage_fire(1, 1, base)

            def win_body(jj, wcarry):
                j0 = 2 * jj
                p0 = drain_scale_fire(0)
                p1 = drain_scale_fire(1)
                for d in p0 + p1:
                    d.wait()

                @pl.when(jj < KW3 // 2 - 1)
                def _():
                    stage_fire(j0 + 2, 0, base)
                    stage_fire(j0 + 3, 1, base)

                return wcarry

            lax.fori_loop(0, KW3 // 2, win_body, 0)
            plsc.subcore_barrier()
            pltpu.sync_copy(acc_sh.at[pl.ds(si * HT, HT)],
                            stage.at[pl.ds(0, HT)])
            pltpu.sync_copy(stage.at[pl.ds(0, HT)],
                            out.at[c, pl.ds(ci * half + si * HT, HT)])
            plsc.subcore_barrier()
            return carry

        lax.fori_loop(0, CH, chunk_body, 0)

    return k


def _silu(x):
    return x / (1.0 + jnp.exp(-x))


def _make_pre_kernel(Np, D1P, H1P, interpret=False):
    GR = Np // NB
    CH = H1P // L

    def body(d0, x16, W1p, dis_ref, sw_ref, xw_ref):
        deg = d0[0, 0, :] + 1.0
        dis = jnp.where(deg > 0,
                        lax.rsqrt(jnp.maximum(deg, 1e-12)),
                        0.0)
        dis_ref[0, 0, :] = dis
        sw_ref[0, 0, :] = dis * dis
        xw = jnp.dot(x16[...], W1p[...], preferred_element_type=jnp.float32)
        for c in range(CH):
            xw_ref[c] = xw[:, c * L:(c + 1) * L]

    return pl.pallas_call(
        body,
        grid=(GR,),
        in_specs=[
            pl.BlockSpec((1, 1, NB), lambda i: (i, 0, 0)),
            pl.BlockSpec((NB, D1P), lambda i: (i, 0)),
            pl.BlockSpec((D1P, H1P), lambda i: (0, 0)),
        ],
        out_specs=[
            pl.BlockSpec((1, 1, NB), lambda i: (i, 0, 0)),
            pl.BlockSpec((1, 1, NB), lambda i: (i, 0, 0)),
            pl.BlockSpec((CH, NB, L), lambda i: (0, i, 0)),
        ],
        out_shape=[
            jax.ShapeDtypeStruct((GR, 1, NB), jnp.float32),
            jax.ShapeDtypeStruct((GR, 1, NB), jnp.float32),
            jax.ShapeDtypeStruct((CH, Np, L), jnp.float32),
        ],
        interpret=interpret,
    )


def _make_layer1_kernel(Np, H1P, H2P, interpret=False):
    GR = Np // NB
    CH = H1P // L
    C2 = H2P // L

    def body(a0, xw1, sw2, b1p, W2p, xw2_ref):
        sw = sw2[0, 0, :]
        conv = jnp.concatenate(
            [a0[c] + xw1[c] * sw[:, None] for c in range(CH)], axis=1)
        h1 = _silu(conv + b1p[...])
        xw2 = jnp.dot(h1, W2p[...], preferred_element_type=jnp.float32)
        for c in range(C2):
            xw2_ref[c] = xw2[:, c * L:(c + 1) * L]

    return pl.pallas_call(
        body,
        grid=(GR,),
        in_specs=[
            pl.BlockSpec((CH, NB, L), lambda i: (0, i, 0)),
            pl.BlockSpec((CH, NB, L), lambda i: (0, i, 0)),
            pl.BlockSpec((1, 1, NB), lambda i: (i, 0, 0)),
            pl.BlockSpec((1, H1P), lambda i: (0, 0)),
            pl.BlockSpec((H1P, H2P), lambda i: (0, 0)),
        ],
        out_specs=pl.BlockSpec((C2, NB, L), lambda i: (0, i, 0)),
        out_shape=jax.ShapeDtypeStruct((C2, Np, L), jnp.float32),
        interpret=interpret,
    )


def _make_final_kernel(Np, H2P, HM, interpret=False):
    GR = Np // NB
    C2 = H2P // L

    def body(a0, xw2, sw2, bat2, b2p, LW1p, Lb1, LW2p, Lb2,
             out_ref, pooled_ref, cnt_ref):
        i = pl.program_id(0)

        @pl.when(i == 0)
        def _():
            pooled_ref[...] = jnp.zeros_like(pooled_ref)
            cnt_ref[...] = jnp.zeros_like(cnt_ref)

        sw = sw2[0, 0, :]
        conv = jnp.concatenate(
            [a0[c] + xw2[c] * sw[:, None] for c in range(C2)], axis=1)
        h2 = _silu(conv + b2p[...])
        b = bat2[0, 0, :]
        seg = lax.broadcasted_iota(jnp.int32, (NB, G), 1)
        onehot = (b[:, None] == seg).astype(jnp.float32)
        pooled_ref[...] += lax.dot_general(
            onehot, h2, (((0,), (0,)), ((), ())),
            preferred_element_type=jnp.float32,
            precision=lax.Precision.HIGHEST)
        cnt_ref[...] += jnp.sum(onehot, axis=0)[None, :]

        @pl.when(i == GR - 1)
        def _():
            cnt = jnp.maximum(cnt_ref[0, :], 1.0)
            pooled = pooled_ref[...] / cnt[:, None]
            hh = _silu(jnp.dot(pooled, LW1p[...],
                               preferred_element_type=jnp.float32) + Lb1[...])
            o = jnp.dot(hh, LW2p[...],
                        preferred_element_type=jnp.float32) + Lb2[0, 0]
            out_ref[...] = o[:, 0][None, :]

    return pl.pallas_call(
        body,
        grid=(GR,),
        in_specs=[
            pl.BlockSpec((C2, NB, L), lambda i: (0, i, 0)),
            pl.BlockSpec((C2, NB, L), lambda i: (0, i, 0)),
            pl.BlockSpec((1, 1, NB), lambda i: (i, 0, 0)),
            pl.BlockSpec((1, 1, NB), lambda i: (i, 0, 0)),
            pl.BlockSpec((1, H2P), lambda i: (0, 0)),
            pl.BlockSpec((H2P, HM), lambda i: (0, 0)),
            pl.BlockSpec((1, HM), lambda i: (0, 0)),
            pl.BlockSpec((HM, 128), lambda i: (0, 0)),
            pl.BlockSpec((1, 1), lambda i: (0, 0)),
        ],
        out_specs=pl.BlockSpec((1, G), lambda i: (0, 0)),
        out_shape=jax.ShapeDtypeStruct((1, G), jnp.float32),
        scratch_shapes=[
            pltpu.VMEM((G, H2P), jnp.float32),
            pltpu.VMEM((1, G), jnp.float32),
        ],
        interpret=interpret,
    )


def kernel(x, edge_index, edge_attr, batch,
           W1, b1, W2, b2, LW1, Lb1, LW2, Lb2):
    N, D = x.shape
    E = edge_index.shape[1]
    H1 = W1.shape[1]
    H2 = W2.shape[1]
    HM = LW1.shape[1]

    Np = _cdiv(N, 2 * NB) * 2 * NB
    KW3 = 2 * _cdiv(E, NS * 1024)
    KW2 = 4 * KW3
    Ep = NS * KW2 * 128
    H1P = _cdiv(H1, L) * L
    H2P = _cdiv(H2, L) * L
    D1P = L

    f32 = jnp.float32
    i32 = jnp.int32

    src = edge_index[0].astype(i32)
    dst = edge_index[1].astype(i32)
    w = edge_attr.astype(f32)
    pad = Ep - E
    spread = (jnp.arange(pad, dtype=i32) % N)
    srcp = jnp.concatenate([src, spread]).reshape(NS, KW2, 128)
    dstp = jnp.concatenate([dst, spread]).reshape(NS, KW2, 128)
    wp = jnp.concatenate([w, jnp.zeros((pad,), f32)]).reshape(NS, KW2, 128)

    x16 = jnp.pad(x.astype(f32), ((0, Np - N), (0, D1P - D)))
    batp = jnp.pad(batch.astype(i32), (0, Np - N), constant_values=-1)
    zeros1 = jnp.zeros((Np,), f32)
    zeros2 = jnp.zeros((Np, L), f32)

    W1p = jnp.pad(W1.astype(f32), ((0, D1P - D), (0, H1P - H1)))
    b1p = jnp.pad(b1.astype(f32), (0, H1P - H1)).reshape(1, H1P)
    W2p = jnp.pad(W2.astype(f32), ((0, H1P - H1), (0, H2P - H2)))
    b2p = jnp.pad(b2.astype(f32), (0, H2P - H2)).reshape(1, H2P)
    LW1p = jnp.pad(LW1.astype(f32), ((0, H2P - H2), (0, 0)))
    Lb1r = Lb1.astype(f32).reshape(1, HM)
    LW2p = jnp.pad(LW2.astype(f32), ((0, 0), (0, 128 - LW2.shape[1])))
    Lb2r = Lb2.astype(f32).reshape(1, 1)

    GR = Np // NB

    deg = _make_deg_kernel(Np, KW2)(dstp, wp, zeros1)

    dis2, sw2, xw1 = _make_pre_kernel(Np, D1P, H1P)(
        deg.reshape(GR, 1, NB), x16, W1p)

    normp = _make_norm_kernel(Np, KW2)(dis2.reshape(Np), srcp, dstp, wp)

    srcp4 = srcp.reshape(NS, KW3, 4, 128)
    dstp4 = dstp.reshape(NS, KW3, 4, 128)
    normp4 = normp.reshape(NS, KW3, 512)

    a1 = _make_prop_kernel(Np, KW3, H1P // L)(
        xw1.reshape((H1P // L) * Np, L), srcp4, dstp4, normp4, zeros2)

    xw2 = _make_layer1_kernel(Np, H1P, H2P)(a1, xw1, sw2, b1p, W2p)

    a2 = _make_prop_kernel(Np, KW3, H2P // L)(
        xw2.reshape((H2P // L) * Np, L), srcp4, dstp4, normp4, zeros2)

    out2 = _make_final_kernel(Np, H2P, HM)(
        a2, xw2, sw2, batp.reshape(GR, 1, NB),
        b2p, LW1p, Lb1r, LW2p, Lb2r)

    return out2.reshape(G)
